# Initial kernel scaffold; baseline (speedup 1.0000x reference)
import jax
import jax.numpy as jnp
from jax.experimental import pallas as pl


def kernel(hiddens, targets, W, b):
    raise NotImplementedError("write your pallas kernel here")


# speedup vs baseline: 2.0924x; 2.0924x over previous
import functools

import jax
import jax.numpy as jnp
from jax.experimental import pallas as pl
from jax.experimental.pallas import tpu as pltpu

_NEG = -1e30


def _loss_body(h_ref, w_ref, b_ref, t_ref, out_ref, m_ref, s_ref, z_ref,
               *, v_total, v_chunk, n_chunks):
    i = pl.program_id(0)
    h = h_ref[:]
    w = w_ref[:]
    chunk = jax.lax.dot_general(
        h, w, (((1,), (1,)), ((), ())),
        preferred_element_type=jnp.float32) + b_ref[:]

    n = h.shape[0]
    vc = w.shape[0]
    col = i * v_chunk + jax.lax.broadcasted_iota(jnp.int32, (n, vc), 1)
    valid = col < v_total
    chunk = jnp.where(valid, chunk, _NEG)

    t = t_ref[:]
    match = col == t
    z_part = jnp.sum(jnp.where(match, chunk, 0.0), axis=1, keepdims=True)

    cmax = jnp.max(chunk, axis=1, keepdims=True)

    @pl.when(i == 0)
    def _init():
        m_ref[:] = cmax
        s_ref[:] = jnp.sum(jnp.exp(chunk - cmax), axis=1, keepdims=True)
        z_ref[:] = z_part

    @pl.when(i > 0)
    def _update():
        m_old = m_ref[:]
        m_new = jnp.maximum(m_old, cmax)
        s_ref[:] = (s_ref[:] * jnp.exp(m_old - m_new)
                    + jnp.sum(jnp.exp(chunk - m_new), axis=1, keepdims=True))
        m_ref[:] = m_new
        z_ref[:] = z_ref[:] + z_part

    @pl.when(i == n_chunks - 1)
    def _final():
        lse = m_ref[:] + jnp.log(s_ref[:])
        wgt = (t != 0).astype(jnp.float32)
        out_ref[:] = jnp.sum(wgt * (lse - z_ref[:]), keepdims=True).reshape(1, 1)


def _nmt_loss(h, t2, w_mat, b2, *, v_chunk=1024, interpret=False):
    n, d = h.shape
    v = w_mat.shape[0]
    n_chunks = pl.cdiv(v, v_chunk)

    body = functools.partial(_loss_body, v_total=v, v_chunk=v_chunk,
                             n_chunks=n_chunks)
    out = pl.pallas_call(
        body,
        grid=(n_chunks,),
        in_specs=[
            pl.BlockSpec((n, d), lambda i: (0, 0)),
            pl.BlockSpec((v_chunk, d), lambda i: (i, 0)),
            pl.BlockSpec((1, v_chunk), lambda i: (0, i)),
            pl.BlockSpec((n, 1), lambda i: (0, 0)),
        ],
        out_specs=pl.BlockSpec((1, 1), lambda i: (0, 0)),
        out_shape=jax.ShapeDtypeStruct((1, 1), jnp.float32),
        scratch_shapes=[
            pltpu.VMEM((n, 1), jnp.float32),
            pltpu.VMEM((n, 1), jnp.float32),
            pltpu.VMEM((n, 1), jnp.float32),
        ],
        compiler_params=pltpu.CompilerParams(
            dimension_semantics=("arbitrary",)),
        interpret=interpret,
    )(h, w_mat, b2, t2)
    return out[0, 0]


def kernel(hiddens, targets, W, b):
    t, bsz, d = hiddens.shape
    h = hiddens.reshape(t * bsz, d)
    t2 = targets.reshape(t * bsz, 1).astype(jnp.int32)
    b2 = b.reshape(1, -1)
    return _nmt_loss(h, t2, W, b2)


# speedup vs baseline: 2.1164x; 1.0115x over previous
import functools

import jax
import jax.numpy as jnp
from jax.experimental import pallas as pl
from jax.experimental.pallas import tpu as pltpu

_NEG = -1e30


def _loss_body(h_ref, w_ref, b_ref, t_ref, out_ref, m_ref, s_ref, z_ref,
               *, v_total, v_chunk, n_chunks):
    i = pl.program_id(0)
    h = h_ref[:]
    w = w_ref[:]
    chunk = jax.lax.dot_general(
        h.astype(jnp.bfloat16), w.astype(jnp.bfloat16),
        (((1,), (1,)), ((), ())),
        preferred_element_type=jnp.float32) + b_ref[:]

    n = h.shape[0]
    vc = w.shape[0]
    col = i * v_chunk + jax.lax.broadcasted_iota(jnp.int32, (n, vc), 1)
    valid = col < v_total
    chunk = jnp.where(valid, chunk, _NEG)

    t = t_ref[:]
    match = col == t
    z_part = jnp.sum(jnp.where(match, chunk, 0.0), axis=1, keepdims=True)

    cmax = jnp.max(chunk, axis=1, keepdims=True)

    @pl.when(i == 0)
    def _init():
        m_ref[:] = cmax
        s_ref[:] = jnp.sum(jnp.exp(chunk - cmax), axis=1, keepdims=True)
        z_ref[:] = z_part

    @pl.when(i > 0)
    def _update():
        m_old = m_ref[:]
        m_new = jnp.maximum(m_old, cmax)
        s_ref[:] = (s_ref[:] * jnp.exp(m_old - m_new)
                    + jnp.sum(jnp.exp(chunk - m_new), axis=1, keepdims=True))
        m_ref[:] = m_new
        z_ref[:] = z_ref[:] + z_part

    @pl.when(i == n_chunks - 1)
    def _final():
        lse = m_ref[:] + jnp.log(s_ref[:])
        wgt = (t != 0).astype(jnp.float32)
        out_ref[:] = jnp.sum(wgt * (lse - z_ref[:]), keepdims=True).reshape(1, 1)


def _nmt_loss(h, t2, w_mat, b2, *, v_chunk=1024, interpret=False):
    n, d = h.shape
    v = w_mat.shape[0]
    n_chunks = pl.cdiv(v, v_chunk)

    body = functools.partial(_loss_body, v_total=v, v_chunk=v_chunk,
                             n_chunks=n_chunks)
    out = pl.pallas_call(
        body,
        grid=(n_chunks,),
        in_specs=[
            pl.BlockSpec((n, d), lambda i: (0, 0)),
            pl.BlockSpec((v_chunk, d), lambda i: (i, 0)),
            pl.BlockSpec((1, v_chunk), lambda i: (0, i)),
            pl.BlockSpec((n, 1), lambda i: (0, 0)),
        ],
        out_specs=pl.BlockSpec((1, 1), lambda i: (0, 0)),
        out_shape=jax.ShapeDtypeStruct((1, 1), jnp.float32),
        scratch_shapes=[
            pltpu.VMEM((n, 1), jnp.float32),
            pltpu.VMEM((n, 1), jnp.float32),
            pltpu.VMEM((n, 1), jnp.float32),
        ],
        compiler_params=pltpu.CompilerParams(
            dimension_semantics=("arbitrary",)),
        interpret=interpret,
    )(h, w_mat, b2, t2)
    return out[0, 0]


def kernel(hiddens, targets, W, b):
    t, bsz, d = hiddens.shape
    h = hiddens.reshape(t * bsz, d)
    t2 = targets.reshape(t * bsz, 1).astype(jnp.int32)
    b2 = b.reshape(1, -1)
    return _nmt_loss(h, t2, W, b2)
